# baseline (device time: 19517 ns/iter reference)
import functools

import jax
import jax.numpy as jnp
from jax import lax
from jax.experimental import pallas as pl
from jax.experimental.pallas import tpu as pltpu

N_DEV = 16
N_CHUNK = 8


def kernel(x):
    m, n = x.shape
    cm = m // N_CHUNK

    def body(x_ref, out_ref, buf_ref, comm_ref, copy_sems, send_sems,
             recv_sems):
        my = lax.axis_index("i")
        barrier = pltpu.get_barrier_semaphore()

        for d in range(1, N_DEV):
            peer = lax.rem(my + d, N_DEV)
            pl.semaphore_signal(
                barrier, inc=1,
                device_id=(peer,), device_id_type=pl.DeviceIdType.MESH,
            )

        def chunk_copy(c, slot):
            return pltpu.make_async_copy(
                x_ref.at[pl.ds(c * cm, cm), :],
                buf_ref.at[slot],
                copy_sems.at[slot],
            )

        chunk_copy(0, 0).start()
        for c in range(N_CHUNK):
            slot = c % 2
            if c + 1 < N_CHUNK:
                chunk_copy(c + 1, (c + 1) % 2).start()
            chunk_copy(c, slot).wait()
            block_max = jnp.max(buf_ref[slot], axis=0, keepdims=True)
            if c == 0:
                comm_ref[my] = block_max
            else:
                comm_ref[my] = jnp.maximum(comm_ref[my], block_max)

        pl.semaphore_wait(barrier, N_DEV - 1)

        for d in range(1, N_DEV):
            dst = lax.rem(my + d, N_DEV)
            rdma = pltpu.make_async_remote_copy(
                src_ref=comm_ref.at[my],
                dst_ref=comm_ref.at[my],
                send_sem=send_sems.at[d - 1],
                recv_sem=recv_sems.at[my],
                device_id=(dst,),
                device_id_type=pl.DeviceIdType.MESH,
            )
            rdma.start()

        for d in range(1, N_DEV):
            src = lax.rem(my + d, N_DEV)
            recv = pltpu.make_async_remote_copy(
                src_ref=comm_ref.at[src],
                dst_ref=comm_ref.at[src],
                send_sem=send_sems.at[d - 1],
                recv_sem=recv_sems.at[src],
                device_id=(src,),
                device_id_type=pl.DeviceIdType.MESH,
            )
            recv.wait_recv()

        for d in range(1, N_DEV):
            dst = lax.rem(my + d, N_DEV)
            snd = pltpu.make_async_remote_copy(
                src_ref=comm_ref.at[my],
                dst_ref=comm_ref.at[my],
                send_sem=send_sems.at[d - 1],
                recv_sem=recv_sems.at[my],
                device_id=(dst,),
                device_id_type=pl.DeviceIdType.MESH,
            )
            snd.wait_send()

        out_ref[...] = jnp.max(comm_ref[...], axis=0)

        @functools.partial(pl.run_scoped, sem=pltpu.SemaphoreType.REGULAR)
        def _(sem):
            for d in range(1, N_DEV):
                peer = lax.rem(my + d, N_DEV)
                pl.semaphore_signal(
                    sem, inc=1,
                    device_id=(peer,), device_id_type=pl.DeviceIdType.MESH,
                )
            pl.semaphore_wait(sem, N_DEV - 1)

    return pl.pallas_call(
        body,
        out_shape=jax.ShapeDtypeStruct((1, n), jnp.float32),
        in_specs=[pl.BlockSpec(memory_space=pl.ANY)],
        out_specs=pl.BlockSpec(memory_space=pltpu.VMEM),
        scratch_shapes=[
            pltpu.VMEM((2, cm, n), jnp.float32),
            pltpu.VMEM((N_DEV, 1, n), jnp.float32),
            pltpu.SemaphoreType.DMA((2,)),
            pltpu.SemaphoreType.DMA((N_DEV,)),
            pltpu.SemaphoreType.DMA((N_DEV,)),
        ],
        compiler_params=pltpu.CompilerParams(collective_id=0),
    )(x)


# device time: 16906 ns/iter; 1.1544x vs baseline; 1.1544x over previous
import functools

import jax
import jax.numpy as jnp
from jax import lax
from jax.experimental import pallas as pl
from jax.experimental.pallas import tpu as pltpu

N_DEV = 16
N_CHUNK = 8


def kernel(x):
    m, n = x.shape
    x = pltpu.with_memory_space_constraint(x, pltpu.MemorySpace.HBM)
    cm = m // N_CHUNK

    def body(x_ref, out_ref, buf_ref, comm_ref, copy_sems, send_sems,
             recv_sems):
        my = lax.axis_index("i")
        barrier = pltpu.get_barrier_semaphore()

        for d in range(1, N_DEV):
            peer = lax.rem(my + d, N_DEV)
            pl.semaphore_signal(
                barrier, inc=1,
                device_id=(peer,), device_id_type=pl.DeviceIdType.MESH,
            )

        def chunk_copy(c, slot):
            return pltpu.make_async_copy(
                x_ref.at[pl.ds(c * cm, cm), :],
                buf_ref.at[slot],
                copy_sems.at[slot],
            )

        chunk_copy(0, 0).start()
        for c in range(N_CHUNK):
            slot = c % 2
            if c + 1 < N_CHUNK:
                chunk_copy(c + 1, (c + 1) % 2).start()
            chunk_copy(c, slot).wait()
            block_max = jnp.max(buf_ref[slot], axis=0, keepdims=True)
            if c == 0:
                comm_ref[my] = block_max
            else:
                comm_ref[my] = jnp.maximum(comm_ref[my], block_max)

        pl.semaphore_wait(barrier, N_DEV - 1)

        for d in range(1, N_DEV):
            dst = lax.rem(my + d, N_DEV)
            rdma = pltpu.make_async_remote_copy(
                src_ref=comm_ref.at[my],
                dst_ref=comm_ref.at[my],
                send_sem=send_sems.at[d - 1],
                recv_sem=recv_sems.at[my],
                device_id=(dst,),
                device_id_type=pl.DeviceIdType.MESH,
            )
            rdma.start()

        for d in range(1, N_DEV):
            src = lax.rem(my + d, N_DEV)
            recv = pltpu.make_async_remote_copy(
                src_ref=comm_ref.at[src],
                dst_ref=comm_ref.at[src],
                send_sem=send_sems.at[d - 1],
                recv_sem=recv_sems.at[src],
                device_id=(src,),
                device_id_type=pl.DeviceIdType.MESH,
            )
            recv.wait_recv()

        for d in range(1, N_DEV):
            dst = lax.rem(my + d, N_DEV)
            snd = pltpu.make_async_remote_copy(
                src_ref=comm_ref.at[my],
                dst_ref=comm_ref.at[my],
                send_sem=send_sems.at[d - 1],
                recv_sem=recv_sems.at[my],
                device_id=(dst,),
                device_id_type=pl.DeviceIdType.MESH,
            )
            snd.wait_send()

        out_ref[...] = jnp.max(comm_ref[...], axis=0)

        @functools.partial(pl.run_scoped, sem=pltpu.SemaphoreType.REGULAR)
        def _(sem):
            for d in range(1, N_DEV):
                peer = lax.rem(my + d, N_DEV)
                pl.semaphore_signal(
                    sem, inc=1,
                    device_id=(peer,), device_id_type=pl.DeviceIdType.MESH,
                )
            pl.semaphore_wait(sem, N_DEV - 1)

    return pl.pallas_call(
        body,
        out_shape=jax.ShapeDtypeStruct((1, n), jnp.float32),
        in_specs=[pl.BlockSpec(memory_space=pltpu.MemorySpace.HBM)],
        out_specs=pl.BlockSpec(memory_space=pltpu.VMEM),
        scratch_shapes=[
            pltpu.VMEM((2, cm, n), jnp.float32),
            pltpu.VMEM((N_DEV, 1, n), jnp.float32),
            pltpu.SemaphoreType.DMA((2,)),
            pltpu.SemaphoreType.DMA((N_DEV,)),
            pltpu.SemaphoreType.DMA((N_DEV,)),
        ],
        compiler_params=pltpu.CompilerParams(collective_id=0),
    )(x)


# device time: 14195 ns/iter; 1.3749x vs baseline; 1.1910x over previous
import functools

import jax
import jax.numpy as jnp
from jax import lax
from jax.experimental import pallas as pl
from jax.experimental.pallas import tpu as pltpu

N_DEV = 16
N_CHUNK = 8


def kernel(x):
    m, n = x.shape
    x = pltpu.with_memory_space_constraint(x, pltpu.MemorySpace.HBM)
    cm = m // N_CHUNK

    def body(x_ref, out_ref, buf_ref, acc_ref, comm_ref, copy_sems,
             send_sems, recv_sems):
        my = lax.axis_index("i")
        barrier = pltpu.get_barrier_semaphore()

        for d in range(1, N_DEV):
            peer = lax.rem(my + d, N_DEV)
            pl.semaphore_signal(
                barrier, inc=1,
                device_id=(peer,), device_id_type=pl.DeviceIdType.MESH,
            )

        copies = [
            pltpu.make_async_copy(
                x_ref.at[pl.ds(c * cm, cm), :],
                buf_ref.at[c],
                copy_sems.at[c],
            )
            for c in range(N_CHUNK)
        ]
        for cp in copies:
            cp.start()
        for c in range(N_CHUNK):
            copies[c].wait()
            strip_max = jnp.max(
                buf_ref[c].reshape(cm // 8, 8, n), axis=0
            )
            if c == 0:
                acc_ref[...] = strip_max
            else:
                acc_ref[...] = jnp.maximum(acc_ref[...], strip_max)
        comm_ref[my] = jnp.max(acc_ref[...], axis=0, keepdims=True)

        pl.semaphore_wait(barrier, N_DEV - 1)

        for d in range(1, N_DEV):
            dst = lax.rem(my + d, N_DEV)
            rdma = pltpu.make_async_remote_copy(
                src_ref=comm_ref.at[my],
                dst_ref=comm_ref.at[my],
                send_sem=send_sems.at[d - 1],
                recv_sem=recv_sems.at[my],
                device_id=(dst,),
                device_id_type=pl.DeviceIdType.MESH,
            )
            rdma.start()

        result = comm_ref[my]
        for d in range(1, N_DEV):
            src = lax.rem(my + d, N_DEV)
            recv = pltpu.make_async_remote_copy(
                src_ref=comm_ref.at[src],
                dst_ref=comm_ref.at[src],
                send_sem=send_sems.at[d - 1],
                recv_sem=recv_sems.at[src],
                device_id=(src,),
                device_id_type=pl.DeviceIdType.MESH,
            )
            recv.wait_recv()
            result = jnp.maximum(result, comm_ref[src])
        out_ref[...] = result

        for d in range(1, N_DEV):
            dst = lax.rem(my + d, N_DEV)
            snd = pltpu.make_async_remote_copy(
                src_ref=comm_ref.at[my],
                dst_ref=comm_ref.at[my],
                send_sem=send_sems.at[d - 1],
                recv_sem=recv_sems.at[my],
                device_id=(dst,),
                device_id_type=pl.DeviceIdType.MESH,
            )
            snd.wait_send()

    return pl.pallas_call(
        body,
        out_shape=jax.ShapeDtypeStruct((1, n), jnp.float32),
        in_specs=[pl.BlockSpec(memory_space=pltpu.MemorySpace.HBM)],
        out_specs=pl.BlockSpec(memory_space=pltpu.VMEM),
        scratch_shapes=[
            pltpu.VMEM((N_CHUNK, cm, n), jnp.float32),
            pltpu.VMEM((8, n), jnp.float32),
            pltpu.VMEM((N_DEV, 1, n), jnp.float32),
            pltpu.SemaphoreType.DMA((N_CHUNK,)),
            pltpu.SemaphoreType.DMA((N_DEV,)),
            pltpu.SemaphoreType.DMA((N_DEV,)),
        ],
        compiler_params=pltpu.CompilerParams(collective_id=0),
    )(x)
